# baseline (speedup 1.0000x reference)
import functools

import jax
import jax.numpy as jnp
from jax import lax
from jax.experimental import pallas as pl
from jax.experimental.pallas import tpu as pltpu
from jax.experimental.pallas import tpu_sc as plsc

_NUM_FIELDS = 26
_VOCAB = 100000
_DIM = 64
_BATCH = 16384

_INFO = plsc.get_sparse_core_info()
_NC = _INFO.num_cores
_NS = _INFO.num_subcores
_NW = _NC * _NS

_SPW = _BATCH // _NW
_G = 16
_NG = _SPW // _G

_MESH = plsc.VectorSubcoreMesh(core_axis_name="c", subcore_axis_name="s")


@functools.partial(
    pl.kernel,
    mesh=_MESH,
    out_type=jax.ShapeDtypeStruct((_BATCH, _NUM_FIELDS, _DIM), jnp.float32),
    scratch_types=[
        pltpu.VMEM((_NUM_FIELDS * _SPW,), jnp.int32),
        pltpu.SemaphoreType.DMA,
    ],
)
def _embed(tables_hbm, idxt_hbm, out_hbm, idx_v, sem_g):
    wid = lax.axis_index("s") * _NC + lax.axis_index("c")
    b0 = pl.multiple_of(wid * _SPW, _SPW)
    for f in range(_NUM_FIELDS):
        pltpu.sync_copy(
            idxt_hbm.at[pl.ds(f * _BATCH + b0, _SPW)],
            idx_v.at[pl.ds(f * _SPW, _SPW)],
        )

    def field(f, carry):
        def group(g, carry2):
            off = pl.multiple_of(f * _SPW + g * _G, _G)
            v = idx_v[pl.ds(off, _G)]
            for k in range(_G):
                t = v[k]
                pltpu.async_copy(
                    tables_hbm.at[f, pl.ds(t, 1), :],
                    out_hbm.at[pl.ds(b0 + g * _G + k, 1), f, :],
                    sem_g,
                )
            return carry2

        lax.fori_loop(0, _NG, group, 0)
        return carry

    lax.fori_loop(0, _NUM_FIELDS, field, 0)

    def drain(i, carry):
        pltpu.make_async_copy(
            tables_hbm.at[0, pl.ds(0, 1), :],
            out_hbm.at[pl.ds(b0, 1), 0, :],
            sem_g,
        ).wait()
        return carry

    lax.fori_loop(0, _NUM_FIELDS * _SPW, drain, 0)


def kernel(x_n_cat, tables):
    idx_t = x_n_cat.T.reshape(-1)
    out3 = _embed(tables, idx_t)
    return out3.reshape(_BATCH, _NUM_FIELDS * _DIM)


# speedup vs baseline: 9.5810x; 9.5810x over previous
import functools

import jax
import jax.numpy as jnp
from jax import lax
from jax.experimental import pallas as pl
from jax.experimental.pallas import tpu as pltpu
from jax.experimental.pallas import tpu_sc as plsc

_NUM_FIELDS = 26
_VOCAB = 100000
_DIM = 64
_BATCH = 16384
_NROWS = _NUM_FIELDS * _DIM
_L = 16

_INFO = plsc.get_sparse_core_info()
_NC = _INFO.num_cores
_NS = _INFO.num_subcores
_NW = _NC * _NS

_RPW = _NROWS // _NW
_HB = _BATCH // 2

_MESH = plsc.VectorSubcoreMesh(core_axis_name="c", subcore_axis_name="s")


@functools.partial(
    pl.kernel,
    mesh=_MESH,
    out_type=jax.ShapeDtypeStruct((_NROWS, _BATCH), jnp.float32),
    scratch_types=[
        pltpu.VMEM((_VOCAB,), jnp.float32),
        pltpu.VMEM((_HB,), jnp.int32),
        pltpu.VMEM((_HB,), jnp.float32),
    ],
    compiler_params=pltpu.CompilerParams(
        use_tc_tiling_on_sc=True, needs_layout_passes=False
    ),
)
def _embed(tabt_hbm, idx2_hbm, out_hbm, row_v, idx_v, col_v):
    wid = lax.axis_index("s") * _NC + lax.axis_index("c")
    fd0 = pl.multiple_of(wid * _RPW, _RPW)

    def row(i, carry):
        fd = fd0 + i
        f = lax.shift_right_logical(fd, 6)
        d = jnp.bitwise_and(fd, _DIM - 1)
        pltpu.sync_copy(tabt_hbm.at[f, d], row_v)

        for h in range(2):
            pltpu.sync_copy(idx2_hbm.at[f, pl.ds(h * _HB, _HB)], idx_v)

            def grp(g, carry2):
                v = plsc.load_gather(row_v, [idx_v[pl.ds(g * _L, _L)]])
                col_v[pl.ds(g * _L, _L)] = v
                return carry2

            lax.fori_loop(0, _HB // _L, grp, 0)
            pltpu.sync_copy(col_v, out_hbm.at[fd, pl.ds(h * _HB, _HB)])
        return carry

    lax.fori_loop(0, _RPW, row, 0)


def kernel(x_n_cat, tables):
    tabt = tables.transpose(0, 2, 1)
    idx2 = x_n_cat.T
    out2 = _embed(tabt, idx2)
    return out2.T


# speedup vs baseline: 10.3701x; 1.0824x over previous
import functools

import jax
import jax.numpy as jnp
from jax import lax
from jax.experimental import pallas as pl
from jax.experimental.pallas import tpu as pltpu
from jax.experimental.pallas import tpu_sc as plsc

_NUM_FIELDS = 26
_VOCAB = 100000
_DIM = 64
_BATCH = 16384
_NROWS = _NUM_FIELDS * _DIM
_L = 16

_INFO = plsc.get_sparse_core_info()
_NC = _INFO.num_cores
_NS = _INFO.num_subcores
_NW = _NC * _NS

_RPW = _NROWS // _NW
_HB = _BATCH // 2
_QB = 4096

_MESH = plsc.VectorSubcoreMesh(core_axis_name="c", subcore_axis_name="s")


@functools.partial(
    pl.kernel,
    mesh=_MESH,
    out_type=jax.ShapeDtypeStruct((_NROWS, _BATCH), jnp.float32),
    scratch_types=[
        pltpu.VMEM((_VOCAB,), jnp.float32),
        pltpu.VMEM((_HB,), jnp.int32),
        pltpu.VMEM((_HB,), jnp.int32),
        pltpu.VMEM((_QB,), jnp.float32),
        pltpu.VMEM((_QB,), jnp.float32),
        pltpu.SemaphoreType.DMA,
        pltpu.SemaphoreType.DMA,
    ],
    compiler_params=pltpu.CompilerParams(
        use_tc_tiling_on_sc=True, needs_layout_passes=False
    ),
)
def _embed(
    tabt_hbm, idx2_hbm, out_hbm, row_v, idxa_v, idxb_v, col0_v, col1_v, s0, s1
):
    wid = lax.axis_index("s") * _NC + lax.axis_index("c")
    fd0 = pl.multiple_of(wid * _RPW, _RPW)
    bufs = (col0_v, col1_v)
    sems = (s0, s1)

    def row(i, carry):
        fd = fd0 + i
        f = lax.shift_right_logical(fd, 6)
        d = jnp.bitwise_and(fd, _DIM - 1)

        @pl.when(jnp.logical_or(i == 0, d == 0))
        def _stage_ids():
            pltpu.sync_copy(idx2_hbm.at[f, pl.ds(0, _HB)], idxa_v)
            pltpu.sync_copy(idx2_hbm.at[f, pl.ds(_HB, _HB)], idxb_v)

        pltpu.sync_copy(tabt_hbm.at[f, d], row_v)

        for h in range(2):
            ids = idxa_v if h == 0 else idxb_v
            for qq in range(2):
                cb = bufs[qq]
                sem = sems[qq]
                if h == 1:
                    pltpu.make_async_copy(
                        cb, out_hbm.at[fd, pl.ds(0, _QB)], sem
                    ).wait()
                else:

                    @pl.when(i > 0)
                    def _reclaim():
                        pltpu.make_async_copy(
                            cb, out_hbm.at[fd, pl.ds(0, _QB)], sem
                        ).wait()

                def grp(g, carry2):
                    v = plsc.load_gather(
                        row_v, [ids[pl.ds(qq * _QB + g * _L, _L)]]
                    )
                    cb[pl.ds(g * _L, _L)] = v
                    return carry2

                lax.fori_loop(0, _QB // _L, grp, 0)
                pltpu.async_copy(
                    cb, out_hbm.at[fd, pl.ds(h * _HB + qq * _QB, _QB)], sem
                )
        return carry

    lax.fori_loop(0, _RPW, row, 0)
    for qq in range(2):
        pltpu.make_async_copy(
            bufs[qq], out_hbm.at[0, pl.ds(0, _QB)], sems[qq]
        ).wait()


def kernel(x_n_cat, tables):
    tabt = tables.transpose(0, 2, 1)
    idx2 = x_n_cat.T
    out2 = _embed(tabt, idx2)
    return out2.T


# speedup vs baseline: 13.7067x; 1.3217x over previous
import functools

import jax
import jax.numpy as jnp
from jax import lax
from jax.experimental import pallas as pl
from jax.experimental.pallas import tpu as pltpu
from jax.experimental.pallas import tpu_sc as plsc

_NUM_FIELDS = 26
_VOCAB = 100000
_DIM = 64
_BATCH = 16384
_NROWS = _NUM_FIELDS * _DIM
_L = 16

_INFO = plsc.get_sparse_core_info()
_NC = _INFO.num_cores
_NS = _INFO.num_subcores
_NW = _NC * _NS

_RPW = _NROWS // _NW
_HB = _BATCH // 2
_QB = 4096

_MESH = plsc.VectorSubcoreMesh(core_axis_name="c", subcore_axis_name="s")


@functools.partial(
    pl.kernel,
    mesh=_MESH,
    out_type=jax.ShapeDtypeStruct((_NROWS, _BATCH), jnp.float32),
    scratch_types=[
        pltpu.VMEM((_VOCAB,), jnp.float32),
        pltpu.VMEM((_HB,), jnp.int32),
        pltpu.VMEM((_HB,), jnp.int32),
        pltpu.VMEM((_QB,), jnp.float32),
        pltpu.VMEM((_QB,), jnp.float32),
        pltpu.SemaphoreType.DMA,
        pltpu.SemaphoreType.DMA,
    ],
    compiler_params=pltpu.CompilerParams(
        use_tc_tiling_on_sc=True, needs_layout_passes=False
    ),
)
def _embed(
    tabt_hbm, idx2_hbm, out_hbm, row_v, idxa_v, idxb_v, col0_v, col1_v, s0, s1
):
    wid = lax.axis_index("s") * _NC + lax.axis_index("c")
    fd0 = pl.multiple_of(wid * _RPW, _RPW)
    bufs = (col0_v, col1_v)
    sems = (s0, s1)

    def row(i, carry):
        fd = fd0 + i
        f = lax.shift_right_logical(fd, 6)
        d = jnp.bitwise_and(fd, _DIM - 1)

        @pl.when(jnp.logical_or(i == 0, d == 0))
        def _stage_ids():
            pltpu.sync_copy(idx2_hbm.at[f, pl.ds(0, _HB)], idxa_v)
            pltpu.sync_copy(idx2_hbm.at[f, pl.ds(_HB, _HB)], idxb_v)

        pltpu.sync_copy(tabt_hbm.at[f, d], row_v)

        for h in range(2):
            ids = idxa_v if h == 0 else idxb_v
            for qq in range(2):
                cb = bufs[qq]
                sem = sems[qq]
                if h == 1:
                    pltpu.make_async_copy(
                        cb, out_hbm.at[fd, pl.ds(0, _QB)], sem
                    ).wait()
                else:

                    @pl.when(i > 0)
                    def _reclaim():
                        pltpu.make_async_copy(
                            cb, out_hbm.at[fd, pl.ds(0, _QB)], sem
                        ).wait()

                def grp(g, carry2):
                    for u in range(8):
                        off = g * (8 * _L) + u * _L
                        v = plsc.load_gather(
                            row_v, [ids[pl.ds(qq * _QB + off, _L)]]
                        )
                        cb[pl.ds(off, _L)] = v
                    return carry2

                lax.fori_loop(0, _QB // (8 * _L), grp, 0)
                pltpu.async_copy(
                    cb, out_hbm.at[fd, pl.ds(h * _HB + qq * _QB, _QB)], sem
                )
        return carry

    lax.fori_loop(0, _RPW, row, 0)
    for qq in range(2):
        pltpu.make_async_copy(
            bufs[qq], out_hbm.at[0, pl.ds(0, _QB)], sems[qq]
        ).wait()


def kernel(x_n_cat, tables):
    tabt = tables.transpose(0, 2, 1)
    idx2 = x_n_cat.T
    out2 = _embed(tabt, idx2)
    return out2.T


# speedup vs baseline: 13.7721x; 1.0048x over previous
import functools

import jax
import jax.numpy as jnp
from jax import lax
from jax.experimental import pallas as pl
from jax.experimental.pallas import tpu as pltpu
from jax.experimental.pallas import tpu_sc as plsc

_NUM_FIELDS = 26
_VOCAB = 100000
_DIM = 64
_BATCH = 16384
_NROWS = _NUM_FIELDS * _DIM
_L = 16

_INFO = plsc.get_sparse_core_info()
_NC = _INFO.num_cores
_NS = _INFO.num_subcores
_NW = _NC * _NS

_RPW = _NROWS // _NW
_HB = _BATCH // 2
_QB = 4096

_MESH = plsc.VectorSubcoreMesh(core_axis_name="c", subcore_axis_name="s")


@functools.partial(
    pl.kernel,
    mesh=_MESH,
    out_type=jax.ShapeDtypeStruct((_NROWS, _BATCH), jnp.float32),
    scratch_types=[
        pltpu.VMEM((_VOCAB,), jnp.float32),
        pltpu.VMEM((_HB,), jnp.int32),
        pltpu.VMEM((_HB,), jnp.int32),
        pltpu.VMEM((_QB,), jnp.float32),
        pltpu.VMEM((_QB,), jnp.float32),
        pltpu.SemaphoreType.DMA,
        pltpu.SemaphoreType.DMA,
    ],
    compiler_params=pltpu.CompilerParams(
        use_tc_tiling_on_sc=True, needs_layout_passes=False
    ),
)
def _embed(
    tabt_hbm, idx2_hbm, out_hbm, row_v, idxa_v, idxb_v, col0_v, col1_v, s0, s1
):
    wid = lax.axis_index("s") * _NC + lax.axis_index("c")
    fd0 = pl.multiple_of(wid * _RPW, _RPW)
    bufs = (col0_v, col1_v)
    sems = (s0, s1)

    def row(i, carry):
        fd = fd0 + i
        f = lax.shift_right_logical(fd, 6)
        d = jnp.bitwise_and(fd, _DIM - 1)

        @pl.when(jnp.logical_or(i == 0, d == 0))
        def _stage_ids():
            pltpu.sync_copy(idx2_hbm.at[f, pl.ds(0, _HB)], idxa_v)
            pltpu.sync_copy(idx2_hbm.at[f, pl.ds(_HB, _HB)], idxb_v)

        pltpu.sync_copy(tabt_hbm.at[f, d], row_v)

        for h in range(2):
            ids = idxa_v if h == 0 else idxb_v
            for qq in range(2):
                cb = bufs[qq]
                sem = sems[qq]
                if h == 1:
                    pltpu.make_async_copy(
                        cb, out_hbm.at[fd, pl.ds(0, _QB)], sem
                    ).wait()
                else:

                    @pl.when(i > 0)
                    def _reclaim():
                        pltpu.make_async_copy(
                            cb, out_hbm.at[fd, pl.ds(0, _QB)], sem
                        ).wait()

                def grp(g, carry2):
                    for u in range(32):
                        off = g * (32 * _L) + u * _L
                        v = plsc.load_gather(
                            row_v, [ids[pl.ds(qq * _QB + off, _L)]]
                        )
                        cb[pl.ds(off, _L)] = v
                    return carry2

                lax.fori_loop(0, _QB // (32 * _L), grp, 0)
                pltpu.async_copy(
                    cb, out_hbm.at[fd, pl.ds(h * _HB + qq * _QB, _QB)], sem
                )
        return carry

    lax.fori_loop(0, _RPW, row, 0)
    for qq in range(2):
        pltpu.make_async_copy(
            bufs[qq], out_hbm.at[0, pl.ds(0, _QB)], sems[qq]
        ).wait()


def kernel(x_n_cat, tables):
    tabt = tables.transpose(0, 2, 1)
    idx2 = x_n_cat.T
    out2 = _embed(tabt, idx2)
    return out2.T
